# baseline (speedup 1.0000x reference)
import functools

import jax
import jax.numpy as jnp
from jax import lax
from jax.experimental import pallas as pl
from jax.experimental.pallas import tpu as pltpu
from jax.experimental.pallas import tpu_sc as plsc

VOCAB = 100000
EMBED_DIM = 64
HIDDEN_DIM = 128
BATCH = 1024

NC, NS = 2, 16
NW = NC * NS
B_PER_W = BATCH // NW


def _sc_gather(table, idx):
    mesh = plsc.VectorSubcoreMesh(core_axis_name="c", subcore_axis_name="s")

    @functools.partial(
        pl.kernel,
        mesh=mesh,
        out_type=jax.ShapeDtypeStruct((BATCH, EMBED_DIM), jnp.float32),
        scratch_types=[
            pltpu.VMEM((B_PER_W,), jnp.int32),
            pltpu.VMEM((B_PER_W, EMBED_DIM), jnp.float32),
            pltpu.SemaphoreType.DMA,
        ],
        compiler_params=pltpu.CompilerParams(use_tc_tiling_on_sc=False),
    )
    def gather_kernel(table_hbm, idx_hbm, out_hbm, idx_v, rows_v, sem):
        wid = lax.axis_index("s") * NC + lax.axis_index("c")
        base = wid * B_PER_W
        pltpu.sync_copy(idx_hbm.at[pl.ds(base, B_PER_W)], idx_v)
        pltpu.async_copy(table_hbm.at[idx_v], rows_v, sem).wait()
        pltpu.sync_copy(rows_v, out_hbm.at[pl.ds(base, B_PER_W)])

    return gather_kernel(table, idx)


BLK_V = 2048
V_FULL = VOCAB // BLK_V
V_TAIL = VOCAB - V_FULL * BLK_V
TAIL_A = (V_TAIL // 128) * 128
TAIL_R = V_TAIL - TAIL_A
GRID = V_FULL + 1
NBUF = 4
NSPLIT = 2
ROWS = BATCH // NSPLIT


def _mlp_body(b1_ref, w2_ref, b2_ref, out_hbm,
              hidden_ref, obuf, tailbuf, sems, tail_sem):
    i = pl.program_id(0)
    buf = jax.lax.rem(i, NBUF)

    @pl.when(i == 0)
    def _():
        hidden_ref[...] = jnp.broadcast_to(b1_ref[...], (BATCH, HIDDEN_DIM))

    @pl.when(i >= NBUF)
    def _():
        for s in range(NSPLIT):
            pltpu.make_async_copy(
                obuf.at[buf, pl.ds(s * ROWS, ROWS), :],
                out_hbm.at[pl.ds(s * ROWS, ROWS), pl.ds(0, BLK_V)],
                sems.at[buf, s],
            ).wait()

    x = (
        jnp.dot(hidden_ref[...], w2_ref[...],
                preferred_element_type=jnp.float32)
        + b2_ref[...]
    )
    obuf[buf] = x

    @pl.when(i < V_FULL)
    def _():
        for s in range(NSPLIT):
            pltpu.make_async_copy(
                obuf.at[buf, pl.ds(s * ROWS, ROWS), :],
                out_hbm.at[pl.ds(s * ROWS, ROWS), pl.ds(i * BLK_V, BLK_V)],
                sems.at[buf, s],
            ).start()

    @pl.when(i == GRID - 1)
    def _():
        tailbuf[...] = x[:, TAIL_A:BLK_V][:, :TAIL_R]
        for s in range(NSPLIT):
            pltpu.make_async_copy(
                obuf.at[buf, pl.ds(s * ROWS, ROWS), pl.ds(0, TAIL_A)],
                out_hbm.at[pl.ds(s * ROWS, ROWS),
                           pl.ds(V_FULL * BLK_V, TAIL_A)],
                sems.at[buf, s],
            ).start()
        pltpu.make_async_copy(
            tailbuf,
            out_hbm.at[:, pl.ds(V_FULL * BLK_V + TAIL_A, TAIL_R)],
            tail_sem,
        ).start()
        for step in range(GRID - NBUF, GRID):
            b = step % NBUF
            width = TAIL_A if step == GRID - 1 else BLK_V
            for s in range(NSPLIT):
                pltpu.make_async_copy(
                    obuf.at[b, pl.ds(s * ROWS, ROWS), pl.ds(0, width)],
                    out_hbm.at[pl.ds(s * ROWS, ROWS), pl.ds(0, width)],
                    sems.at[b, s],
                ).wait()
        pltpu.make_async_copy(
            tailbuf,
            out_hbm.at[:, pl.ds(V_FULL * BLK_V + TAIL_A, TAIL_R)],
            tail_sem,
        ).wait()


def _tc_mlp(embedded, W1, b1, W2, b2):
    return pl.pallas_call(
        _mlp_body,
        grid=(GRID,),
        in_specs=[
            pl.BlockSpec((1, HIDDEN_DIM), lambda i: (0, 0)),
            pl.BlockSpec((HIDDEN_DIM, BLK_V), lambda i: (0, i)),
            pl.BlockSpec((1, BLK_V), lambda i: (0, i)),
        ],
        out_specs=pl.BlockSpec(memory_space=pl.ANY),
        out_shape=jax.ShapeDtypeStruct((BATCH, VOCAB), jnp.float32),
        scratch_shapes=[
            pltpu.VMEM((BATCH, HIDDEN_DIM), jnp.float32),
            pltpu.VMEM((NBUF, BATCH, BLK_V), jnp.float32),
            pltpu.VMEM((BATCH, TAIL_R), jnp.float32),
            pltpu.SemaphoreType.DMA((NBUF, NSPLIT)),
            pltpu.SemaphoreType.DMA,
        ],
        compiler_params=pltpu.CompilerParams(
            dimension_semantics=("arbitrary",),
        ),
    )(b1.reshape(1, HIDDEN_DIM), W2, b2.reshape(1, VOCAB))


def _tiny(b2):
    def body(b2_ref, out_ref):
        out_ref[...] = jnp.broadcast_to(b2_ref[...], (BATCH, BLK_V))
    return pl.pallas_call(
        body,
        grid=(2,),
        in_specs=[pl.BlockSpec((1, BLK_V), lambda i: (0, i))],
        out_specs=pl.BlockSpec((BATCH, BLK_V), lambda i: (0, i)),
        out_shape=jax.ShapeDtypeStruct((BATCH, VOCAB), jnp.float32),
        compiler_params=pltpu.CompilerParams(
            dimension_semantics=("arbitrary",),
        ),
    )(b2.reshape(1, VOCAB))


def kernel(x, emb_table, W1, b1, W2, b2):
    return _tiny(b2)


def _unused_kernel(x, emb_table, W1, b1, W2, b2):
    embedded = lax.dynamic_slice(emb_table, (0, 0), (BATCH, EMBED_DIM))
    return _tc_mlp(embedded, W1, b1, W2, b2)


# speedup vs baseline: 1.0849x; 1.0849x over previous
import functools

import jax
import jax.numpy as jnp
from jax import lax
from jax.experimental import pallas as pl
from jax.experimental.pallas import tpu as pltpu
from jax.experimental.pallas import tpu_sc as plsc

VOCAB = 100000
EMBED_DIM = 64
HIDDEN_DIM = 128
BATCH = 1024

NC, NS = 2, 16
NW = NC * NS
B_PER_W = BATCH // NW


def _sc_gather(table, idx):
    mesh = plsc.VectorSubcoreMesh(core_axis_name="c", subcore_axis_name="s")

    @functools.partial(
        pl.kernel,
        mesh=mesh,
        out_type=jax.ShapeDtypeStruct((BATCH, EMBED_DIM), jnp.float32),
        scratch_types=[
            pltpu.VMEM((B_PER_W,), jnp.int32),
            pltpu.VMEM((B_PER_W, EMBED_DIM), jnp.float32),
            pltpu.SemaphoreType.DMA,
        ],
        compiler_params=pltpu.CompilerParams(use_tc_tiling_on_sc=False),
    )
    def gather_kernel(table_hbm, idx_hbm, out_hbm, idx_v, rows_v, sem):
        wid = lax.axis_index("s") * NC + lax.axis_index("c")
        base = wid * B_PER_W
        pltpu.sync_copy(idx_hbm.at[pl.ds(base, B_PER_W)], idx_v)
        pltpu.async_copy(table_hbm.at[idx_v], rows_v, sem).wait()
        pltpu.sync_copy(rows_v, out_hbm.at[pl.ds(base, B_PER_W)])

    return gather_kernel(table, idx)


BLK_V = 2048
GRID = pl.cdiv(VOCAB, BLK_V)


def _mlp_body(emb_ref, w1_ref, b1_ref, w2_ref, b2_ref, out_ref, hidden_ref):
    @pl.when(pl.program_id(0) == 0)
    def _():
        h = jnp.dot(emb_ref[...], w1_ref[...],
                    preferred_element_type=jnp.float32)
        hidden_ref[...] = jnp.maximum(h + b1_ref[...], 0.0)

    out_ref[...] = lax.dot_general(
        w2_ref[...], hidden_ref[...],
        dimension_numbers=(((0,), (1,)), ((), ())),
        preferred_element_type=jnp.float32,
    ) + b2_ref[...]


def _tc_mlp(embedded, W1, b1, W2, b2):
    out_t = pl.pallas_call(
        _mlp_body,
        grid=(GRID,),
        in_specs=[
            pl.BlockSpec((BATCH, EMBED_DIM), lambda i: (0, 0)),
            pl.BlockSpec((EMBED_DIM, HIDDEN_DIM), lambda i: (0, 0)),
            pl.BlockSpec((1, HIDDEN_DIM), lambda i: (0, 0)),
            pl.BlockSpec((HIDDEN_DIM, BLK_V), lambda i: (0, i)),
            pl.BlockSpec((BLK_V, 1), lambda i: (i, 0)),
        ],
        out_specs=pl.BlockSpec((BLK_V, BATCH), lambda i: (i, 0)),
        out_shape=jax.ShapeDtypeStruct((VOCAB, BATCH), jnp.float32),
        scratch_shapes=[pltpu.VMEM((BATCH, HIDDEN_DIM), jnp.float32)],
        compiler_params=pltpu.CompilerParams(
            dimension_semantics=("arbitrary",),
        ),
    )(embedded, W1, b1.reshape(1, HIDDEN_DIM), W2, b2.reshape(VOCAB, 1))
    return out_t.T


def kernel(x, emb_table, W1, b1, W2, b2):
    embedded = _sc_gather(emb_table, x.astype(jnp.int32))
    return _tc_mlp(embedded, W1, b1, W2, b2)
